# baseline (speedup 1.0000x reference)
import functools

import jax
import jax.numpy as jnp
from jax import lax
from jax.experimental import pallas as pl
from jax.experimental.pallas import tpu as pltpu
from jax.experimental.pallas import tpu_sc as plsc

B, T, D = 4096, 50, 64
TP = 64
BB = 256
M = BB * TP
NPH = 1
BP = B // NPH

_NC, _NS = 2, 16
_NW = _NC * _NS
_BEH_W = BP * TP // _NW
_CHUNK = 512
_NCHUNK = _BEH_W // _CHUNK
_CANDW = BP // _NW


def _sc_gather(table, idx_beh, idx_cand, cat_table, cat_idx):
    mesh = plsc.VectorSubcoreMesh(core_axis_name="c", subcore_axis_name="s")

    @functools.partial(
        pl.kernel,
        out_type=(
            jax.ShapeDtypeStruct((BP * TP, D), jnp.float32),
            jax.ShapeDtypeStruct((BP, D), jnp.float32),
            jax.ShapeDtypeStruct((BP, D // 2), jnp.float32),
        ),
        mesh=mesh,
        scratch_types=[
            pltpu.VMEM((_BEH_W,), jnp.int32),
            pltpu.VMEM((_CHUNK, D), jnp.float32),
            pltpu.VMEM((_CHUNK, D), jnp.float32),
            pltpu.VMEM((_CANDW,), jnp.int32),
            pltpu.VMEM((_CANDW, D), jnp.float32),
            pltpu.VMEM((_CANDW,), jnp.int32),
            pltpu.VMEM((_CANDW, D // 2), jnp.float32),
            pltpu.SemaphoreType.DMA,
            pltpu.SemaphoreType.DMA,
        ],
        compiler_params=pltpu.CompilerParams(use_tc_tiling_on_sc=False),
    )
    def k(table_hbm, ibeh_hbm, icand_hbm, ctab_hbm, cidx_hbm,
          beh_hbm, cand_hbm, cat_hbm,
          idx_v, rows_a, rows_b, kidx_v, krows_v, cidx_v, crows_v,
          sem_a, sem_b):
        wid = lax.axis_index("s") * _NC + lax.axis_index("c")
        base = wid * _BEH_W
        pltpu.sync_copy(ibeh_hbm.at[pl.ds(base, _BEH_W)], idx_v)
        bufs = (rows_a, rows_b)
        sems = (sem_a, sem_b)
        copies = [None, None]
        copies[0] = pltpu.async_copy(
            table_hbm.at[idx_v.at[pl.ds(0, _CHUNK)]], rows_a, sem_a)
        for i in range(_NCHUNK):
            if i + 1 < _NCHUNK:
                copies[(i + 1) % 2] = pltpu.async_copy(
                    table_hbm.at[idx_v.at[pl.ds((i + 1) * _CHUNK, _CHUNK)]],
                    bufs[(i + 1) % 2], sems[(i + 1) % 2])
            copies[i % 2].wait()
            pltpu.sync_copy(bufs[i % 2],
                            beh_hbm.at[pl.ds(base + i * _CHUNK, _CHUNK)])
        kbase = wid * _CANDW
        pltpu.sync_copy(icand_hbm.at[pl.ds(kbase, _CANDW)], kidx_v)
        pltpu.async_copy(table_hbm.at[kidx_v], krows_v, sem_a).wait()
        pltpu.sync_copy(krows_v, cand_hbm.at[pl.ds(kbase, _CANDW)])
        pltpu.sync_copy(cidx_hbm.at[pl.ds(kbase, _CANDW)], cidx_v)
        pltpu.async_copy(ctab_hbm.at[cidx_v], crows_v, sem_b).wait()
        pltpu.sync_copy(crows_v, cat_hbm.at[pl.ds(kbase, _CANDW)])

    return k(table, idx_beh, idx_cand, cat_table, cat_idx)


def _tc_body(be_ref, cand_ref, cat_ref, dense_ref,
             ad_ref, b0_ref, cm_ref, w1t_ref, b1_ref, w2r_ref,
             dw_ref, db_ref,
             m0_ref, mb0_ref, g0_ref, lb0_ref,
             m1_ref, mb1_ref, g1_ref, lb1_ref,
             m2_ref, mb2_ref, g2_ref, lb2_ref,
             hw_ref, hb_ref, sc_ref, out_ref):
    a0 = sc_ref[0, 0]
    a1 = sc_ref[0, 1]
    am = (sc_ref[0, 2], sc_ref[0, 3], sc_ref[0, 4])
    b2 = sc_ref[0, 5]

    be2 = be_ref[...]
    be3 = be2.reshape(BB, TP, D)
    cand = cand_ref[...]
    cand2 = jnp.broadcast_to(cand[:, None, :], (BB, TP, D)).reshape(M, D)
    prod = be2 * cand2

    candC = jnp.dot(cand, cm_ref[...], preferred_element_type=jnp.float32)
    candC2 = jnp.broadcast_to(candC[:, None, :], (BB, TP, 64)).reshape(M, 64)

    ad = ad_ref[...]
    h0 = (jnp.dot(be2, ad[0:D], preferred_element_type=jnp.float32)
          + jnp.dot(prod, ad[D:2 * D], preferred_element_type=jnp.float32)
          + candC2 + b0_ref[...])
    h0 = jnp.where(h0 >= 0, h0, a0 * h0)
    h1 = jnp.dot(h0, w1t_ref[...], preferred_element_type=jnp.float32) + b1_ref[...]
    h1 = jnp.where(h1 >= 0, h1, a1 * h1)
    s = jnp.dot(h1, w2r_ref[...],
                preferred_element_type=jnp.float32) + b2

    absum = jnp.sum(jnp.abs(be2), axis=1, keepdims=True)
    m = jnp.where(absum > 0, jnp.float32(1.0), jnp.float32(0.0))
    sm = s.reshape(BB, TP, 1)
    tmask = jnp.where(
        jax.lax.broadcasted_iota(jnp.int32, (BB, TP, 1), 1) < T,
        jnp.float32(1.0), jnp.float32(0.0))
    m3 = m.reshape(BB, TP, 1) * tmask
    mx = jnp.max(sm, axis=1, keepdims=True)
    e = jnp.exp(sm - mx) * m3
    den = jnp.sum(e, axis=1, keepdims=True)
    w3 = e / den
    ui = jnp.sum(w3 * be3, axis=1)

    dense_out = jnp.dot(dense_ref[...], dw_ref[...],
                        preferred_element_type=jnp.float32) + db_ref[...]
    x2 = jnp.concatenate([ui, cand, cat_ref[...], dense_out], axis=1)

    mlps = ((m0_ref, mb0_ref, g0_ref, lb0_ref),
            (m1_ref, mb1_ref, g1_ref, lb1_ref),
            (m2_ref, mb2_ref, g2_ref, lb2_ref))
    for li, (mw, mb, g, lb) in enumerate(mlps):
        x2 = jnp.dot(x2, mw[...], preferred_element_type=jnp.float32) + mb[...]
        mu = jnp.mean(x2, axis=1, keepdims=True)
        var = jnp.mean((x2 - mu) * (x2 - mu), axis=1, keepdims=True)
        x2 = (x2 - mu) * lax.rsqrt(var + 1e-5) * g[...] + lb[...]
        x2 = jnp.where(x2 >= 0, x2, am[li] * x2)

    z = jnp.dot(x2, hw_ref[...], preferred_element_type=jnp.float32) + hb_ref[...]
    out_ref[...] = 1.0 / (1.0 + jnp.exp(-z))


def _bcast(i):
    return 0, 0


_TC_IN_SPECS = [
    pl.BlockSpec((M, D), lambda i: (i, 0)),
    pl.BlockSpec((BB, D), lambda i: (i, 0)),
    pl.BlockSpec((BB, D // 2), lambda i: (i, 0)),
    pl.BlockSpec((BB, 8), lambda i: (i, 0)),
    pl.BlockSpec((2 * D, 64), _bcast),
    pl.BlockSpec((1, 64), _bcast),
    pl.BlockSpec((D, D), _bcast),
    pl.BlockSpec((64, 32), _bcast),
    pl.BlockSpec((1, 32), _bcast),
    pl.BlockSpec((32, 1), _bcast),
    pl.BlockSpec((8, 32), _bcast),
    pl.BlockSpec((1, 32), _bcast),
    pl.BlockSpec((192, 256), _bcast),
    pl.BlockSpec((1, 256), _bcast),
    pl.BlockSpec((1, 256), _bcast),
    pl.BlockSpec((1, 256), _bcast),
    pl.BlockSpec((256, 128), _bcast),
    pl.BlockSpec((1, 128), _bcast),
    pl.BlockSpec((1, 128), _bcast),
    pl.BlockSpec((1, 128), _bcast),
    pl.BlockSpec((128, 64), _bcast),
    pl.BlockSpec((1, 64), _bcast),
    pl.BlockSpec((1, 64), _bcast),
    pl.BlockSpec((1, 64), _bcast),
    pl.BlockSpec((64, 8), _bcast),
    pl.BlockSpec((1, 8), _bcast),
    pl.BlockSpec((1, 8), _bcast),
]

_TC_OUT_SPEC = pl.BlockSpec((BB, 8), lambda i: (i, 0))


def _tc_forward(ops, interpret=False):
    return pl.pallas_call(
        _tc_body,
        grid=(BP // BB,),
        in_specs=_TC_IN_SPECS,
        out_specs=_TC_OUT_SPEC,
        out_shape=jax.ShapeDtypeStruct((BP, 8), jnp.float32),
        compiler_params=pltpu.CompilerParams(
            dimension_semantics=("parallel",)),
        interpret=interpret,
    )(*ops)


def _prep_weights(p):
    w0t = p['att_w0'].T
    A = w0t[0:D] + w0t[2 * D:3 * D]
    Cm = w0t[D:2 * D] - w0t[2 * D:3 * D]
    Dm = w0t[3 * D:4 * D]
    AD = jnp.concatenate([A, Dm], axis=0)
    dwp = jnp.pad(p['dense_w'].T, ((0, 3), (0, 0)))
    hw = jnp.pad(
        jnp.stack([p[f'head{t}_w'][0] for t in range(3)], axis=1),
        ((0, 0), (0, 5)))
    hb = jnp.pad(
        jnp.stack([p[f'head{t}_b'][0] for t in range(3)])[None, :],
        ((0, 0), (0, 5)))
    scal = jnp.stack([
        p['att_a0'], p['att_a1'], p['mlp0_a'], p['mlp1_a'], p['mlp2_a'],
        p['att_b2'][0], jnp.float32(0), jnp.float32(0)])[None, :]
    return (
        AD, p['att_b0'][None, :], Cm,
        p['att_w1'].T, p['att_b1'][None, :], p['att_w2'].T,
        dwp, p['dense_b'][None, :],
        p['mlp0_w'].T, p['mlp0_b'][None, :], p['mlp0_lng'][None, :], p['mlp0_lnb'][None, :],
        p['mlp1_w'].T, p['mlp1_b'][None, :], p['mlp1_lng'][None, :], p['mlp1_lnb'][None, :],
        p['mlp2_w'].T, p['mlp2_b'][None, :], p['mlp2_lng'][None, :], p['mlp2_lnb'][None, :],
        hw, hb, scal,
    )


def kernel(behavior_ids, candidate_id, candidate_cat, dense_features, params):
    p = params
    ids = behavior_ids.astype(jnp.int32)
    zpos = 100000 + (
        jax.lax.broadcasted_iota(jnp.int32, (B, T), 0) * T
        + jax.lax.broadcasted_iota(jnp.int32, (B, T), 1)) % 2048
    idx_real = jnp.where(ids == 0, zpos, ids)
    fill = (jax.lax.broadcasted_iota(jnp.int32, (B, TP - T), 0) * (TP - T)
            + jax.lax.broadcasted_iota(jnp.int32, (B, TP - T), 1)) % 100000
    idx_beh = jnp.concatenate([idx_real, fill], axis=1)
    tablez = jnp.concatenate(
        [p['item_emb'], jnp.zeros((2048, D), jnp.float32)], axis=0)
    cand_i = candidate_id.astype(jnp.int32)
    cat_i = candidate_cat.astype(jnp.int32)
    weights = _prep_weights(p)

    outs = []
    for ph in range(NPH):
        lo = ph * BP
        be, cand, cat_rows = _sc_gather(
            tablez, idx_beh[lo:lo + BP].reshape(-1),
            cand_i[lo:lo + BP], p['cat_emb'], cat_i[lo:lo + BP])
        ops = (be, cand, cat_rows,
               jnp.pad(dense_features[lo:lo + BP], ((0, 0), (0, 3)))) + weights
        outs.append(_tc_forward(ops))
    return jnp.concatenate(outs, axis=0)[:, :3]


# speedup vs baseline: 1.0183x; 1.0183x over previous
import functools

import jax
import jax.numpy as jnp
from jax import lax
from jax.experimental import pallas as pl
from jax.experimental.pallas import tpu as pltpu
from jax.experimental.pallas import tpu_sc as plsc

B, T, D = 4096, 50, 64
TP = 64
BB = 256
M = BB * TP
NPH = 1
BP = B // NPH

_NC, _NS = 2, 16
_NW = _NC * _NS
_BEH_W = BP * TP // _NW
_CHUNK = 512
_NCHUNK = _BEH_W // _CHUNK
_CANDW = BP // _NW


def _sc_gather(table, idx_beh, idx_cand, cat_table, cat_idx):
    mesh = plsc.VectorSubcoreMesh(core_axis_name="c", subcore_axis_name="s")

    @functools.partial(
        pl.kernel,
        out_type=(
            jax.ShapeDtypeStruct((BP * TP, D), jnp.float32),
            jax.ShapeDtypeStruct((BP, D), jnp.float32),
            jax.ShapeDtypeStruct((BP, D // 2), jnp.float32),
        ),
        mesh=mesh,
        scratch_types=[
            pltpu.VMEM((_BEH_W,), jnp.int32),
            pltpu.VMEM((_CHUNK, D), jnp.float32),
            pltpu.VMEM((_CHUNK, D), jnp.float32),
            pltpu.VMEM((_CANDW,), jnp.int32),
            pltpu.VMEM((_CANDW, D), jnp.float32),
            pltpu.VMEM((_CANDW,), jnp.int32),
            pltpu.VMEM((_CANDW, D // 2), jnp.float32),
            pltpu.SemaphoreType.DMA,
            pltpu.SemaphoreType.DMA,
        ],
        compiler_params=pltpu.CompilerParams(use_tc_tiling_on_sc=False),
    )
    def k(table_hbm, ibeh_hbm, icand_hbm, ctab_hbm, cidx_hbm,
          beh_hbm, cand_hbm, cat_hbm,
          idx_v, rows_a, rows_b, kidx_v, krows_v, cidx_v, crows_v,
          sem_a, sem_b):
        wid = lax.axis_index("s") * _NC + lax.axis_index("c")
        base = wid * _BEH_W
        pltpu.sync_copy(ibeh_hbm.at[pl.ds(base, _BEH_W)], idx_v)
        bufs = (rows_a, rows_b)
        sems = (sem_a, sem_b)
        copies = [None, None]
        copies[0] = pltpu.async_copy(
            table_hbm.at[idx_v.at[pl.ds(0, _CHUNK)]], rows_a, sem_a)
        for i in range(_NCHUNK):
            if i + 1 < _NCHUNK:
                copies[(i + 1) % 2] = pltpu.async_copy(
                    table_hbm.at[idx_v.at[pl.ds((i + 1) * _CHUNK, _CHUNK)]],
                    bufs[(i + 1) % 2], sems[(i + 1) % 2])
            copies[i % 2].wait()
            pltpu.sync_copy(bufs[i % 2],
                            beh_hbm.at[pl.ds(base + i * _CHUNK, _CHUNK)])
        kbase = wid * _CANDW
        pltpu.sync_copy(icand_hbm.at[pl.ds(kbase, _CANDW)], kidx_v)
        pltpu.async_copy(table_hbm.at[kidx_v], krows_v, sem_a).wait()
        pltpu.sync_copy(krows_v, cand_hbm.at[pl.ds(kbase, _CANDW)])
        pltpu.sync_copy(cidx_hbm.at[pl.ds(kbase, _CANDW)], cidx_v)
        pltpu.async_copy(ctab_hbm.at[cidx_v], crows_v, sem_b).wait()
        pltpu.sync_copy(crows_v, cat_hbm.at[pl.ds(kbase, _CANDW)])

    return k(table, idx_beh, idx_cand, cat_table, cat_idx)


def _tc_body(be_ref, cand_ref, cat_ref, dense_ref,
             ad_ref, b0_ref, cm_ref, w1t_ref, b1_ref, w2r_ref,
             dw_ref, db_ref,
             m0_ref, mb0_ref, g0_ref, lb0_ref,
             m1_ref, mb1_ref, g1_ref, lb1_ref,
             m2_ref, mb2_ref, g2_ref, lb2_ref,
             hw_ref, hb_ref, sc_ref, out_ref):
    a0 = sc_ref[0, 0]
    a1 = sc_ref[0, 1]
    am = (sc_ref[0, 2], sc_ref[0, 3], sc_ref[0, 4])
    b2 = sc_ref[0, 5]

    be2 = be_ref[...]
    be3 = be2.reshape(BB, TP, D)
    cand = cand_ref[...]
    cand2 = jnp.broadcast_to(cand[:, None, :], (BB, TP, D)).reshape(M, D)
    prod = be2 * cand2

    candC = jnp.dot(cand, cm_ref[...], preferred_element_type=jnp.float32)
    candC2 = jnp.broadcast_to(candC[:, None, :], (BB, TP, 64)).reshape(M, 64)

    x = jnp.concatenate([be2, prod], axis=1)
    h0 = (jnp.dot(x, ad_ref[...], preferred_element_type=jnp.float32)
          + candC2 + b0_ref[...])
    h0 = jnp.where(h0 >= 0, h0, a0 * h0)
    h1 = jnp.dot(h0, w1t_ref[...], preferred_element_type=jnp.float32) + b1_ref[...]
    h1 = jnp.where(h1 >= 0, h1, a1 * h1)
    s = jnp.dot(h1, w2r_ref[...],
                preferred_element_type=jnp.float32) + b2

    absum = jnp.sum(jnp.abs(be2), axis=1, keepdims=True)
    m = jnp.where(absum > 0, jnp.float32(1.0), jnp.float32(0.0))
    sm = s.reshape(BB, TP, 1)
    tmask = jnp.where(
        jax.lax.broadcasted_iota(jnp.int32, (BB, TP, 1), 1) < T,
        jnp.float32(1.0), jnp.float32(0.0))
    m3 = m.reshape(BB, TP, 1) * tmask
    mx = jnp.max(sm, axis=1, keepdims=True)
    e = jnp.exp(sm - mx) * m3
    den = jnp.sum(e, axis=1, keepdims=True)
    w3 = e / den
    ui = jnp.sum(w3 * be3, axis=1)

    dense_out = jnp.dot(dense_ref[...], dw_ref[...],
                        preferred_element_type=jnp.float32) + db_ref[...]
    x2 = jnp.concatenate([ui, cand, cat_ref[...], dense_out], axis=1)

    mlps = ((m0_ref, mb0_ref, g0_ref, lb0_ref),
            (m1_ref, mb1_ref, g1_ref, lb1_ref),
            (m2_ref, mb2_ref, g2_ref, lb2_ref))
    for li, (mw, mb, g, lb) in enumerate(mlps):
        x2 = jnp.dot(x2, mw[...], preferred_element_type=jnp.float32) + mb[...]
        mu = jnp.mean(x2, axis=1, keepdims=True)
        var = jnp.mean((x2 - mu) * (x2 - mu), axis=1, keepdims=True)
        x2 = (x2 - mu) * lax.rsqrt(var + 1e-5) * g[...] + lb[...]
        x2 = jnp.where(x2 >= 0, x2, am[li] * x2)

    z = jnp.dot(x2, hw_ref[...], preferred_element_type=jnp.float32) + hb_ref[...]
    out_ref[...] = 1.0 / (1.0 + jnp.exp(-z))


def _bcast(i):
    return 0, 0


_TC_IN_SPECS = [
    pl.BlockSpec((M, D), lambda i: (i, 0)),
    pl.BlockSpec((BB, D), lambda i: (i, 0)),
    pl.BlockSpec((BB, D // 2), lambda i: (i, 0)),
    pl.BlockSpec((BB, 8), lambda i: (i, 0)),
    pl.BlockSpec((2 * D, 64), _bcast),
    pl.BlockSpec((1, 64), _bcast),
    pl.BlockSpec((D, D), _bcast),
    pl.BlockSpec((64, 32), _bcast),
    pl.BlockSpec((1, 32), _bcast),
    pl.BlockSpec((32, 1), _bcast),
    pl.BlockSpec((8, 32), _bcast),
    pl.BlockSpec((1, 32), _bcast),
    pl.BlockSpec((192, 256), _bcast),
    pl.BlockSpec((1, 256), _bcast),
    pl.BlockSpec((1, 256), _bcast),
    pl.BlockSpec((1, 256), _bcast),
    pl.BlockSpec((256, 128), _bcast),
    pl.BlockSpec((1, 128), _bcast),
    pl.BlockSpec((1, 128), _bcast),
    pl.BlockSpec((1, 128), _bcast),
    pl.BlockSpec((128, 64), _bcast),
    pl.BlockSpec((1, 64), _bcast),
    pl.BlockSpec((1, 64), _bcast),
    pl.BlockSpec((1, 64), _bcast),
    pl.BlockSpec((64, 8), _bcast),
    pl.BlockSpec((1, 8), _bcast),
    pl.BlockSpec((1, 8), _bcast),
]

_TC_OUT_SPEC = pl.BlockSpec((BB, 8), lambda i: (i, 0))


def _tc_forward(ops, interpret=False):
    return pl.pallas_call(
        _tc_body,
        grid=(BP // BB,),
        in_specs=_TC_IN_SPECS,
        out_specs=_TC_OUT_SPEC,
        out_shape=jax.ShapeDtypeStruct((BP, 8), jnp.float32),
        compiler_params=pltpu.CompilerParams(
            dimension_semantics=("parallel",)),
        interpret=interpret,
    )(*ops)


def _prep_weights(p):
    w0t = p['att_w0'].T
    A = w0t[0:D] + w0t[2 * D:3 * D]
    Cm = w0t[D:2 * D] - w0t[2 * D:3 * D]
    Dm = w0t[3 * D:4 * D]
    AD = jnp.concatenate([A, Dm], axis=0)
    dwp = jnp.pad(p['dense_w'].T, ((0, 3), (0, 0)))
    hw = jnp.pad(
        jnp.stack([p[f'head{t}_w'][0] for t in range(3)], axis=1),
        ((0, 0), (0, 5)))
    hb = jnp.pad(
        jnp.stack([p[f'head{t}_b'][0] for t in range(3)])[None, :],
        ((0, 0), (0, 5)))
    scal = jnp.stack([
        p['att_a0'], p['att_a1'], p['mlp0_a'], p['mlp1_a'], p['mlp2_a'],
        p['att_b2'][0], jnp.float32(0), jnp.float32(0)])[None, :]
    return (
        AD, p['att_b0'][None, :], Cm,
        p['att_w1'].T, p['att_b1'][None, :], p['att_w2'].T,
        dwp, p['dense_b'][None, :],
        p['mlp0_w'].T, p['mlp0_b'][None, :], p['mlp0_lng'][None, :], p['mlp0_lnb'][None, :],
        p['mlp1_w'].T, p['mlp1_b'][None, :], p['mlp1_lng'][None, :], p['mlp1_lnb'][None, :],
        p['mlp2_w'].T, p['mlp2_b'][None, :], p['mlp2_lng'][None, :], p['mlp2_lnb'][None, :],
        hw, hb, scal,
    )


def kernel(behavior_ids, candidate_id, candidate_cat, dense_features, params):
    p = params
    ids = behavior_ids.astype(jnp.int32)
    zpos = 100000 + (
        jax.lax.broadcasted_iota(jnp.int32, (B, T), 0) * T
        + jax.lax.broadcasted_iota(jnp.int32, (B, T), 1)) % 2048
    idx_real = jnp.where(ids == 0, zpos, ids)
    fill = (jax.lax.broadcasted_iota(jnp.int32, (B, TP - T), 0) * (TP - T)
            + jax.lax.broadcasted_iota(jnp.int32, (B, TP - T), 1)) % 100000
    idx_beh = jnp.concatenate([idx_real, fill], axis=1)
    tablez = jnp.concatenate(
        [p['item_emb'], jnp.zeros((2048, D), jnp.float32)], axis=0)
    cand_i = candidate_id.astype(jnp.int32)
    cat_i = candidate_cat.astype(jnp.int32)
    weights = _prep_weights(p)

    outs = []
    for ph in range(NPH):
        lo = ph * BP
        be, cand, cat_rows = _sc_gather(
            tablez, idx_beh[lo:lo + BP].reshape(-1),
            cand_i[lo:lo + BP], p['cat_emb'], cat_i[lo:lo + BP])
        ops = (be, cand, cat_rows,
               jnp.pad(dense_features[lo:lo + BP], ((0, 0), (0, 3)))) + weights
        outs.append(_tc_forward(ops))
    return jnp.concatenate(outs, axis=0)[:, :3]
